# baseline (speedup 1.0000x reference)
import functools

import jax
import jax.numpy as jnp
from jax import lax
from jax.experimental import pallas as pl
from jax.experimental.pallas import tpu as pltpu
from jax.experimental.pallas import tpu_sc as plsc

N = 10000
E = 320000
C = 128

NPAD = 10240
NCORES = 2
NSUB = 16
EPT = E // NSUB
CHUNK = 64
NPROC = 4 * (-(-EPT // (4 * CHUNK)))
NALLOC = NPROC + 4
EPT_PAD = NALLOC * CHUNK
ROWS_PT = NPAD // NSUB
DRAIN = 64
NDRAIN = ROWS_PT // DRAIN
LANES = 16
SCAT_BYTES = CHUNK * C * 4


def _mm_body(x_ref, wc_ref, wl_ref, xm_ref, xid_ref):
    x = x_ref[...]
    xm_ref[...] = jnp.dot(x, wc_ref[...], preferred_element_type=jnp.float32)
    xid_ref[...] = jnp.dot(x, wl_ref[...], preferred_element_type=jnp.float32)


def _matmuls(x, w_conv, w_lin):
    blk = 2048
    return pl.pallas_call(
        _mm_body,
        grid=(NPAD // blk,),
        in_specs=[
            pl.BlockSpec((blk, C), lambda i: (i, 0)),
            pl.BlockSpec((C, C), lambda i: (0, 0)),
            pl.BlockSpec((C, C), lambda i: (0, 0)),
        ],
        out_specs=[
            pl.BlockSpec((blk, C), lambda i: (i, 0)),
            pl.BlockSpec((blk, C), lambda i: (i, 0)),
        ],
        out_shape=[
            jax.ShapeDtypeStruct((NPAD, C), jnp.float32),
            jax.ShapeDtypeStruct((NPAD, C), jnp.float32),
        ],
    )(x, w_conv, w_lin)


def _combine_body(o_ref, xid_ref, out_ref):
    out_ref[...] = jax.nn.sigmoid(o_ref[0] + o_ref[1] - xid_ref[...])


def _combine(o, x_id):
    blk = 2000
    return pl.pallas_call(
        _combine_body,
        grid=(N // blk,),
        in_specs=[
            pl.BlockSpec((NCORES, blk, C), lambda i: (0, i, 0)),
            pl.BlockSpec((blk, C), lambda i: (i, 0)),
        ],
        out_specs=pl.BlockSpec((blk, C), lambda i: (i, 0)),
        out_shape=jax.ShapeDtypeStruct((N, C), jnp.float32),
    )(o, x_id)


def _splat(v16, lane):
    idx = jnp.full((LANES,), lane, dtype=jnp.int32)
    return lax.gather(
        v16,
        idx[:, None],
        dimension_numbers=lax.GatherDimensionNumbers(
            offset_dims=(), collapsed_slice_dims=(0,), start_index_map=(0,)
        ),
        slice_sizes=(1,),
        mode=lax.GatherScatterMode.PROMISE_IN_BOUNDS,
    )


@functools.partial(
    pl.kernel,
    out_type=jax.ShapeDtypeStruct((NCORES, NPAD, C), jnp.float32),
    mesh=plsc.VectorSubcoreMesh(core_axis_name="c", subcore_axis_name="s"),
    scratch_types=(
        [pltpu.VMEM((2, CHUNK), jnp.int32)] * 4
        + [pltpu.VMEM((CHUNK,), jnp.float32)] * 4
        + [pltpu.VMEM((CHUNK,), jnp.int32)] * 4
        + [pltpu.VMEM((CHUNK, C), jnp.float32)] * 4
        + [pltpu.VMEM((DRAIN, C), jnp.float32)]
        + [pltpu.VMEM_SHARED((NPAD, C), jnp.float32)]
        + [pltpu.SemaphoreType.DMA] * 12
    ),
)
def _sc_spmm(xm_hbm, xid_hbm, epack_hbm, vpack_hbm, out_hbm,
             e0, e1, e2, e3, v0, v1, v2, v3, r0, r1, r2, r3,
             g0, g1, g2, g3, xbuf, acc,
             sE0, sE1, sE2, sE3, sG0, sG1, sG2, sG3, sS0, sS1, sS2, sS3):
    c = lax.axis_index("c")
    s = lax.axis_index("s")
    base = s * ROWS_PT
    ebuf = [e0, e1, e2, e3]
    vbuf = [v0, v1, v2, v3]
    rbuf = [r0, r1, r2, r3]
    gath = [g0, g1, g2, g3]
    semE = [sE0, sE1, sE2, sE3]
    semG = [sG0, sG1, sG2, sG3]
    semS = [sS0, sS1, sS2, sS3]

    for i in range(NDRAIN):
        r_0 = base + i * DRAIN
        pltpu.sync_copy(xid_hbm.at[pl.ds(r_0, DRAIN), :], xbuf)
        pltpu.sync_copy(xbuf, acc.at[pl.ds(r_0, DRAIN), :])
    plsc.subcore_barrier()

    def _scale(g_, v_):
        for gg in range(CHUNK // LANES):
            vals16 = v_[pl.ds(gg * LANES, LANES)]
            for e in range(LANES):
                v = _splat(vals16, e)
                row = gg * LANES + e
                for cc in range(C // LANES):
                    sl = pl.ds(cc * LANES, LANES)
                    g_[row, sl] = g_[row, sl] * v

    def _eload(j, i):
        pltpu.async_copy(epack_hbm.at[c, s, j], ebuf[i], semE[i])
        pltpu.async_copy(vpack_hbm.at[c, s, j], vbuf[i], semE[i])

    def _ewait(i):
        pltpu.make_async_copy(epack_hbm.at[c, s, 0], ebuf[i], semE[i]).wait()
        pltpu.make_async_copy(vpack_hbm.at[c, s, 0], vbuf[i], semE[i]).wait()

    H = CHUNK // 2

    def _gissue(i):
        cols = ebuf[i].at[1]
        pltpu.async_copy(xm_hbm.at[cols.at[pl.ds(0, H)]],
                         gath[i].at[pl.ds(0, H)], semG[i])
        pltpu.async_copy(xm_hbm.at[cols.at[pl.ds(H, H)]],
                         gath[i].at[pl.ds(H, H)], semG[i])

    def _gwait(i):
        cols = ebuf[i].at[1]
        pltpu.make_async_copy(xm_hbm.at[cols.at[pl.ds(0, H)]],
                              gath[i].at[pl.ds(0, H)], semG[i]).wait()
        pltpu.make_async_copy(xm_hbm.at[cols.at[pl.ds(H, H)]],
                              gath[i].at[pl.ds(H, H)], semG[i]).wait()

    def _swait(i):
        pltpu.make_async_copy(gath[i], acc.at[rbuf[i]], semS[i]).wait()

    for i in range(4):
        _eload(i, i)
    zf = jnp.zeros((LANES,), jnp.float32)
    zi = jnp.zeros((LANES,), jnp.int32)
    for i in (3,):
        for row in range(CHUNK):
            for cc in range(C // LANES):
                gath[i][row, pl.ds(cc * LANES, LANES)] = zf
        for gg in range(CHUNK // LANES):
            rbuf[i][pl.ds(gg * LANES, LANES)] = zi
        pltpu.async_copy(gath[i], acc.at[rbuf[i]], semS[i], add=True)
    _ewait(0)
    _gissue(0)
    _ewait(1)
    _gissue(1)
    _ewait(2)
    _gissue(2)

    def quad_body(b, carry):
        j0 = 4 * b
        for i in range(4):
            k = (i + 3) % 4
            _gwait(i)
            _scale(gath[i], vbuf[i])
            for gg in range(CHUNK // LANES):
                sl = pl.ds(gg * LANES, LANES)
                rbuf[i][sl] = ebuf[i][0, sl]
            pltpu.async_copy(gath[i], acc.at[rbuf[i]], semS[i], add=True)
            _eload(j0 + i + 4, i)
            _swait(k)
            _ewait(k)
            _gissue(k)
        return carry

    lax.fori_loop(0, NPROC // 4, quad_body, 0)
    _swait(3)
    _gwait(0)
    _gwait(1)
    _gwait(2)
    _ewait(3)
    plsc.subcore_barrier()

    for i in range(NDRAIN):
        r_0 = base + i * DRAIN
        pltpu.sync_copy(acc.at[pl.ds(r_0, DRAIN), :], xbuf)
        pltpu.sync_copy(xbuf, out_hbm.at[c, pl.ds(r_0, DRAIN), :])


def _pack_edges(edge_index, values):
    pad = ((0, 0), (0, EPT_PAD - EPT))
    rows = jnp.pad(edge_index[0].astype(jnp.int32).reshape(NSUB, EPT), pad)
    cols = jnp.pad(edge_index[1].astype(jnp.int32).reshape(NSUB, EPT), pad)
    vals = jnp.pad(values.astype(jnp.float32).reshape(NSUB, EPT), pad)
    idx = jnp.stack(
        [rows.reshape(NSUB, NALLOC, CHUNK),
         cols.reshape(NSUB, NALLOC, CHUNK)], axis=2)
    return idx, vals.reshape(NSUB, NALLOC, CHUNK)


def kernel(x, down_edge_index, down_values, up_edge_index, up_values,
           W_conv, W_lin):
    x_p = jnp.pad(x, ((0, NPAD - N), (0, 0)))
    xm, x_id = _matmuls(x_p, W_conv, W_lin)
    di, dv = _pack_edges(down_edge_index, down_values)
    ui, uv = _pack_edges(up_edge_index, up_values)
    o = _sc_spmm(xm, x_id, jnp.stack([di, ui]), jnp.stack([dv, uv]))
    return _combine(o, x_id)


# speedup vs baseline: 1.0619x; 1.0619x over previous
import functools

import jax
import jax.numpy as jnp
from jax import lax
from jax.experimental import pallas as pl
from jax.experimental.pallas import tpu as pltpu
from jax.experimental.pallas import tpu_sc as plsc

N = 10000
E = 320000
C = 128

NPAD = 10240
NCORES = 2
NSUB = 16
EPT = E // NSUB
CHUNK = 64
NPROC = 4 * (-(-EPT // (4 * CHUNK)))
NALLOC = NPROC + 4
EPT_PAD = NALLOC * CHUNK
ROWS_PT = NPAD // NSUB
DRAIN = 64
NDRAIN = ROWS_PT // DRAIN
LANES = 16
SCAT_BYTES = CHUNK * C * 4


def _mm_body(x_ref, wc_ref, wl_ref, xm_ref, xid_ref):
    x = x_ref[...]
    xm_ref[...] = jnp.dot(x, wc_ref[...], preferred_element_type=jnp.float32)
    xid_ref[...] = jnp.dot(x, wl_ref[...], preferred_element_type=jnp.float32)


def _matmuls(x, w_conv, w_lin):
    blk = 2048
    return pl.pallas_call(
        _mm_body,
        grid=(NPAD // blk,),
        in_specs=[
            pl.BlockSpec((blk, C), lambda i: (i, 0)),
            pl.BlockSpec((C, C), lambda i: (0, 0)),
            pl.BlockSpec((C, C), lambda i: (0, 0)),
        ],
        out_specs=[
            pl.BlockSpec((blk, C), lambda i: (i, 0)),
            pl.BlockSpec((blk, C), lambda i: (i, 0)),
        ],
        out_shape=[
            jax.ShapeDtypeStruct((NPAD, C), jnp.float32),
            jax.ShapeDtypeStruct((NPAD, C), jnp.float32),
        ],
    )(x, w_conv, w_lin)


def _combine_body(o_ref, xid_ref, out_ref):
    out_ref[...] = jax.nn.sigmoid(o_ref[0] + o_ref[1] - xid_ref[...])


def _combine(o, x_id):
    blk = 2000
    return pl.pallas_call(
        _combine_body,
        grid=(N // blk,),
        in_specs=[
            pl.BlockSpec((NCORES, blk, C), lambda i: (0, i, 0)),
            pl.BlockSpec((blk, C), lambda i: (i, 0)),
        ],
        out_specs=pl.BlockSpec((blk, C), lambda i: (i, 0)),
        out_shape=jax.ShapeDtypeStruct((N, C), jnp.float32),
    )(o, x_id)


def _splat(v16, lane):
    idx = jnp.full((LANES,), lane, dtype=jnp.int32)
    return lax.gather(
        v16,
        idx[:, None],
        dimension_numbers=lax.GatherDimensionNumbers(
            offset_dims=(), collapsed_slice_dims=(0,), start_index_map=(0,)
        ),
        slice_sizes=(1,),
        mode=lax.GatherScatterMode.PROMISE_IN_BOUNDS,
    )


@functools.partial(
    pl.kernel,
    out_type=jax.ShapeDtypeStruct((NCORES, NPAD, C), jnp.float32),
    mesh=plsc.VectorSubcoreMesh(core_axis_name="c", subcore_axis_name="s"),
    scratch_types=(
        [pltpu.VMEM((3, CHUNK), jnp.float32)] * 4
        + [pltpu.VMEM((CHUNK,), jnp.int32)] * 4
        + [pltpu.VMEM((CHUNK,), jnp.int32)] * 4
        + [pltpu.VMEM((CHUNK, C), jnp.float32)] * 4
        + [pltpu.VMEM((DRAIN, C), jnp.float32)]
        + [pltpu.VMEM_SHARED((NPAD, C), jnp.float32)]
        + [pltpu.SemaphoreType.DMA] * 12
    ),
)
def _sc_spmm(xm_hbm, xid_hbm, epack_hbm, out_hbm,
             e0, e1, e2, e3, cb0, cb1, cb2, cb3, r0, r1, r2, r3,
             g0, g1, g2, g3, xbuf, acc,
             sE0, sE1, sE2, sE3, sG0, sG1, sG2, sG3, sS0, sS1, sS2, sS3):
    c = lax.axis_index("c")
    s = lax.axis_index("s")
    base = s * ROWS_PT
    ebuf = [e0, e1, e2, e3]
    cbuf = [cb0, cb1, cb2, cb3]
    rbuf = [r0, r1, r2, r3]
    gath = [g0, g1, g2, g3]
    semE = [sE0, sE1, sE2, sE3]
    semG = [sG0, sG1, sG2, sG3]
    semS = [sS0, sS1, sS2, sS3]

    for i in range(NDRAIN):
        r_0 = base + i * DRAIN
        pltpu.sync_copy(xid_hbm.at[pl.ds(r_0, DRAIN), :], xbuf)
        pltpu.sync_copy(xbuf, acc.at[pl.ds(r_0, DRAIN), :])
    plsc.subcore_barrier()

    def _scale(g_, v_):
        for gg in range(CHUNK // LANES):
            vals16 = v_[2, pl.ds(gg * LANES, LANES)]
            for e in range(LANES):
                v = _splat(vals16, e)
                row = gg * LANES + e
                for cc in range(C // LANES):
                    sl = pl.ds(cc * LANES, LANES)
                    g_[row, sl] = g_[row, sl] * v

    def _eload(j, i):
        pltpu.async_copy(epack_hbm.at[c, s, j], ebuf[i], semE[i])

    def _ewait(i):
        pltpu.make_async_copy(epack_hbm.at[c, s, 0], ebuf[i], semE[i]).wait()
        for gg in range(CHUNK // LANES):
            sl = pl.ds(gg * LANES, LANES)
            rbuf[i][sl] = ebuf[i][0, sl].astype(jnp.int32)
            cbuf[i][sl] = ebuf[i][1, sl].astype(jnp.int32)

    def _gissue(i):
        pltpu.async_copy(xm_hbm.at[cbuf[i]], gath[i], semG[i])

    def _gwait(i):
        pltpu.make_async_copy(xm_hbm.at[cbuf[i]], gath[i],
                              semG[i]).wait()

    def _swait(i):
        pltpu.make_async_copy(gath[i], acc.at[rbuf[i]], semS[i]).wait()

    for i in range(4):
        _eload(i, i)
    zf = jnp.zeros((LANES,), jnp.float32)
    zi = jnp.zeros((LANES,), jnp.int32)
    for i in (2, 3):
        for row in range(CHUNK):
            for cc in range(C // LANES):
                gath[i][row, pl.ds(cc * LANES, LANES)] = zf
        for gg in range(CHUNK // LANES):
            rbuf[i][pl.ds(gg * LANES, LANES)] = zi
        pltpu.async_copy(gath[i], acc.at[rbuf[i]], semS[i], add=True)
    _ewait(0)
    _gissue(0)
    _ewait(1)
    _gissue(1)

    def quad_body(b, carry):
        j0 = 4 * b
        for i in range(4):
            k = (i + 2) % 4
            _gwait(i)
            _scale(gath[i], ebuf[i])
            pltpu.async_copy(gath[i], acc.at[rbuf[i]], semS[i], add=True)
            _eload(j0 + i + 4, i)
            _swait(k)
            _ewait(k)
            _gissue(k)
        return carry

    lax.fori_loop(0, NPROC // 4, quad_body, 0)
    _swait(2)
    _swait(3)
    _gwait(0)
    _gwait(1)
    _ewait(2)
    _ewait(3)
    plsc.subcore_barrier()

    for i in range(NDRAIN):
        r_0 = base + i * DRAIN
        pltpu.sync_copy(acc.at[pl.ds(r_0, DRAIN), :], xbuf)
        pltpu.sync_copy(xbuf, out_hbm.at[c, pl.ds(r_0, DRAIN), :])


def _pack_edges(edge_index, values):
    pad = ((0, 0), (0, EPT_PAD - EPT))
    rows = jnp.pad(
        edge_index[0].astype(jnp.float32).reshape(NSUB, EPT), pad)
    cols = jnp.pad(
        edge_index[1].astype(jnp.float32).reshape(NSUB, EPT), pad)
    vals = jnp.pad(values.astype(jnp.float32).reshape(NSUB, EPT), pad)
    return jnp.stack(
        [rows.reshape(NSUB, NALLOC, CHUNK),
         cols.reshape(NSUB, NALLOC, CHUNK),
         vals.reshape(NSUB, NALLOC, CHUNK)], axis=2)


def kernel(x, down_edge_index, down_values, up_edge_index, up_values,
           W_conv, W_lin):
    x_p = jnp.pad(x, ((0, NPAD - N), (0, 0)))
    xm, x_id = _matmuls(x_p, W_conv, W_lin)
    o = _sc_spmm(xm, x_id,
                 jnp.stack([_pack_edges(down_edge_index, down_values),
                            _pack_edges(up_edge_index, up_values)]))
    return _combine(o, x_id)
